# baseline (speedup 1.0000x reference)
import jax
import jax.numpy as jnp
from jax.experimental import pallas as pl
from jax.experimental.pallas import tpu as pltpu

D_IN = 4096
RANK = 64
N_CODES = 16

BM = 512
NSLOTS = 4
NCHUNK = 2
CH = D_IN // NCHUNK


def _dequant(idx, codebook_row):
    b0 = (idx & 1) == 1
    b1 = (idx & 2) == 2
    b2 = (idx & 4) == 4
    b3 = (idx & 8) == 8
    v = [jnp.where(b0, codebook_row[0, 2 * p + 1], codebook_row[0, 2 * p])
         for p in range(8)]
    w = [jnp.where(b1, v[2 * q + 1], v[2 * q]) for q in range(4)]
    x = [jnp.where(b2, w[1], w[0]), jnp.where(b2, w[3], w[2])]
    return jnp.where(b3, x[1], x[0]).astype(jnp.bfloat16)


def _out_copy(obuf_ref, hbm_out_ref, sem, step, slot):
    return pltpu.make_async_copy(
        obuf_ref.at[slot],
        hbm_out_ref.at[pl.ds(step * BM, BM), :],
        sem.at[slot],
    )


def _band_dot(a, b):
    return jax.lax.dot_general(
        a, b, (((1,), (0,)), ((), ())),
        preferred_element_type=jnp.float32,
        precision=jax.lax.Precision.DEFAULT,
    )


def _fused_kernel(a_idx_hbm, b_idx_hbm, ca_hbm, cb_hbm, hbm_out_ref,
                  a_idx_ref, b_idx_ref, ca_ref, cb_ref,
                  a_deq_ref, b_deq_ref, obuf_ref, sem, in_sem):
    i = pl.program_id(0)
    n = pl.num_programs(0)
    slot = jax.lax.rem(i, NSLOTS)

    @pl.when(i == 0)
    def _():
        copies = (
            pltpu.make_async_copy(a_idx_hbm, a_idx_ref, in_sem.at[0]),
            pltpu.make_async_copy(b_idx_hbm, b_idx_ref, in_sem.at[1]),
            pltpu.make_async_copy(ca_hbm, ca_ref, in_sem.at[2]),
            pltpu.make_async_copy(cb_hbm, cb_ref, in_sem.at[3]),
        )
        for c in copies:
            c.start()
        for c in copies:
            c.wait()

    @pl.when(i >= NSLOTS)
    def _():
        _out_copy(obuf_ref, hbm_out_ref, sem, i - NSLOTS, slot).wait()

    a_deq_ref[...] = _dequant(a_idx_ref[pl.ds(i * BM, BM), :], ca_ref[...])
    a = a_deq_ref[...]

    @pl.when(i == 0)
    def _():
        for c in range(NCHUNK):
            sl = slice(c * CH, (c + 1) * CH)
            b_deq_ref[:, sl] = _dequant(b_idx_ref[:, sl], cb_ref[...])
            obuf_ref[0, :, sl] = _band_dot(a, b_deq_ref[:, sl])

    @pl.when(i > 0)
    def _():
        obuf_ref[slot] = _band_dot(a, b_deq_ref[...])

    _out_copy(obuf_ref, hbm_out_ref, sem, i, slot).start()

    @pl.when(i == n - 1)
    def _():
        for d in range(NSLOTS - 1, -1, -1):
            @pl.when(i - d >= 0)
            def _():
                _out_copy(obuf_ref, hbm_out_ref, sem, i - d,
                          jax.lax.rem(i - d, NSLOTS)).wait()


def _run_local(a_idx, b_idx, ca, cb):
    d_out_local = a_idx.shape[0]
    return pl.pallas_call(
        _fused_kernel,
        grid=(d_out_local // BM,),
        in_specs=[
            pl.BlockSpec(memory_space=pl.ANY),
            pl.BlockSpec(memory_space=pl.ANY),
            pl.BlockSpec(memory_space=pl.ANY),
            pl.BlockSpec(memory_space=pl.ANY),
        ],
        out_specs=pl.BlockSpec(memory_space=pl.ANY),
        out_shape=jax.ShapeDtypeStruct((d_out_local, D_IN), jnp.float32),
        scratch_shapes=[
            pltpu.VMEM((d_out_local, RANK), jnp.int32),
            pltpu.VMEM((RANK, D_IN), jnp.int32),
            pltpu.VMEM((1, N_CODES), jnp.float32),
            pltpu.VMEM((1, N_CODES), jnp.float32),
            pltpu.VMEM((BM, RANK), jnp.bfloat16),
            pltpu.VMEM((RANK, D_IN), jnp.bfloat16),
            pltpu.VMEM((NSLOTS, BM, D_IN), jnp.float32),
            pltpu.SemaphoreType.DMA((NSLOTS,)),
            pltpu.SemaphoreType.DMA((4,)),
        ],
        compiler_params=pltpu.CompilerParams(
            dimension_semantics=("arbitrary",),
        ),
    )(a_idx, b_idx, ca, cb)


def kernel(A_assignments, B_assignments, A_codebook, B_codebook):
    ca = A_codebook.reshape(1, N_CODES).astype(jnp.float32)
    cb = B_codebook.reshape(1, N_CODES).astype(jnp.float32)
    return _run_local(A_assignments, B_assignments, ca, cb)


# speedup vs baseline: 1.0169x; 1.0169x over previous
import jax
import jax.numpy as jnp
from jax.experimental import pallas as pl
from jax.experimental.pallas import tpu as pltpu

D_IN = 4096
RANK = 64
N_CODES = 16

BM = 256
NSLOTS = 4
NCHUNK = 4
CH = D_IN // NCHUNK


def _dequant(idx, codebook_row):
    b0 = (idx & 1) == 1
    b1 = (idx & 2) == 2
    b2 = (idx & 4) == 4
    b3 = (idx & 8) == 8
    v = [jnp.where(b0, codebook_row[0, 2 * p + 1], codebook_row[0, 2 * p])
         for p in range(8)]
    w = [jnp.where(b1, v[2 * q + 1], v[2 * q]) for q in range(4)]
    x = [jnp.where(b2, w[1], w[0]), jnp.where(b2, w[3], w[2])]
    return jnp.where(b3, x[1], x[0]).astype(jnp.bfloat16)


def _out_copy(obuf_ref, hbm_out_ref, sem, step, slot):
    return pltpu.make_async_copy(
        obuf_ref.at[slot],
        hbm_out_ref.at[pl.ds(step * BM, BM), :],
        sem.at[slot],
    )


def _band_dot(a, b):
    return jax.lax.dot_general(
        a, b, (((1,), (0,)), ((), ())),
        preferred_element_type=jnp.float32,
        precision=jax.lax.Precision.DEFAULT,
    )


def _fused_kernel(a_idx_hbm, b_idx_hbm, ca_hbm, cb_hbm, hbm_out_ref,
                  a_idx_ref, b_idx_ref, ca_ref, cb_ref,
                  a_deq_ref, b_deq_ref, obuf_ref, sem, in_sem):
    i = pl.program_id(0)
    n = pl.num_programs(0)
    slot = jax.lax.rem(i, NSLOTS)

    @pl.when(i == 0)
    def _():
        copies = (
            pltpu.make_async_copy(a_idx_hbm, a_idx_ref, in_sem.at[0]),
            pltpu.make_async_copy(b_idx_hbm, b_idx_ref, in_sem.at[1]),
            pltpu.make_async_copy(ca_hbm, ca_ref, in_sem.at[2]),
            pltpu.make_async_copy(cb_hbm, cb_ref, in_sem.at[3]),
        )
        for c in copies:
            c.start()
        for c in copies:
            c.wait()

    @pl.when(i >= NSLOTS)
    def _():
        _out_copy(obuf_ref, hbm_out_ref, sem, i - NSLOTS, slot).wait()

    a_deq_ref[...] = _dequant(a_idx_ref[pl.ds(i * BM, BM), :], ca_ref[...])
    a = a_deq_ref[...]

    @pl.when(i == 0)
    def _():
        for c in range(NCHUNK):
            sl = slice(c * CH, (c + 1) * CH)
            b_deq_ref[:, sl] = _dequant(b_idx_ref[:, sl], cb_ref[...])
            obuf_ref[0, :, sl] = _band_dot(a, b_deq_ref[:, sl])

    @pl.when(i > 0)
    def _():
        obuf_ref[slot] = _band_dot(a, b_deq_ref[...])

    _out_copy(obuf_ref, hbm_out_ref, sem, i, slot).start()

    @pl.when(i == n - 1)
    def _():
        for d in range(NSLOTS - 1, -1, -1):
            @pl.when(i - d >= 0)
            def _():
                _out_copy(obuf_ref, hbm_out_ref, sem, i - d,
                          jax.lax.rem(i - d, NSLOTS)).wait()


def _run_local(a_idx, b_idx, ca, cb):
    d_out_local = a_idx.shape[0]
    return pl.pallas_call(
        _fused_kernel,
        grid=(d_out_local // BM,),
        in_specs=[
            pl.BlockSpec(memory_space=pl.ANY),
            pl.BlockSpec(memory_space=pl.ANY),
            pl.BlockSpec(memory_space=pl.ANY),
            pl.BlockSpec(memory_space=pl.ANY),
        ],
        out_specs=pl.BlockSpec(memory_space=pl.ANY),
        out_shape=jax.ShapeDtypeStruct((d_out_local, D_IN), jnp.float32),
        scratch_shapes=[
            pltpu.VMEM((d_out_local, RANK), jnp.int32),
            pltpu.VMEM((RANK, D_IN), jnp.int32),
            pltpu.VMEM((1, N_CODES), jnp.float32),
            pltpu.VMEM((1, N_CODES), jnp.float32),
            pltpu.VMEM((BM, RANK), jnp.bfloat16),
            pltpu.VMEM((RANK, D_IN), jnp.bfloat16),
            pltpu.VMEM((NSLOTS, BM, D_IN), jnp.float32),
            pltpu.SemaphoreType.DMA((NSLOTS,)),
            pltpu.SemaphoreType.DMA((4,)),
        ],
        compiler_params=pltpu.CompilerParams(
            dimension_semantics=("arbitrary",),
        ),
    )(a_idx, b_idx, ca, cb)


def kernel(A_assignments, B_assignments, A_codebook, B_codebook):
    ca = A_codebook.reshape(1, N_CODES).astype(jnp.float32)
    cb = B_codebook.reshape(1, N_CODES).astype(jnp.float32)
    return _run_local(A_assignments, B_assignments, ca, cb)


# speedup vs baseline: 1.0201x; 1.0032x over previous
import jax
import jax.numpy as jnp
from jax.experimental import pallas as pl
from jax.experimental.pallas import tpu as pltpu

D_IN = 4096
RANK = 64
N_CODES = 16

BM = 128
NSLOTS = 4
NCHUNK = 4
CH = D_IN // NCHUNK


def _dequant(idx, codebook_row):
    b0 = (idx & 1) == 1
    b1 = (idx & 2) == 2
    b2 = (idx & 4) == 4
    b3 = (idx & 8) == 8
    v = [jnp.where(b0, codebook_row[0, 2 * p + 1], codebook_row[0, 2 * p])
         for p in range(8)]
    w = [jnp.where(b1, v[2 * q + 1], v[2 * q]) for q in range(4)]
    x = [jnp.where(b2, w[1], w[0]), jnp.where(b2, w[3], w[2])]
    return jnp.where(b3, x[1], x[0]).astype(jnp.bfloat16)


def _out_copy(obuf_ref, hbm_out_ref, sem, step, slot):
    return pltpu.make_async_copy(
        obuf_ref.at[slot],
        hbm_out_ref.at[pl.ds(step * BM, BM), :],
        sem.at[slot],
    )


def _band_dot(a, b):
    return jax.lax.dot_general(
        a, b, (((1,), (0,)), ((), ())),
        preferred_element_type=jnp.float32,
        precision=jax.lax.Precision.DEFAULT,
    )


def _fused_kernel(a_idx_hbm, b_idx_hbm, ca_hbm, cb_hbm, hbm_out_ref,
                  a_idx_ref, b_idx_ref, ca_ref, cb_ref,
                  a_deq_ref, b_deq_ref, obuf_ref, sem, in_sem):
    i = pl.program_id(0)
    n = pl.num_programs(0)
    slot = jax.lax.rem(i, NSLOTS)

    @pl.when(i == 0)
    def _():
        copies = (
            pltpu.make_async_copy(a_idx_hbm, a_idx_ref, in_sem.at[0]),
            pltpu.make_async_copy(b_idx_hbm, b_idx_ref, in_sem.at[1]),
            pltpu.make_async_copy(ca_hbm, ca_ref, in_sem.at[2]),
            pltpu.make_async_copy(cb_hbm, cb_ref, in_sem.at[3]),
        )
        for c in copies:
            c.start()
        for c in copies:
            c.wait()

    @pl.when(i >= NSLOTS)
    def _():
        _out_copy(obuf_ref, hbm_out_ref, sem, i - NSLOTS, slot).wait()

    a_deq_ref[...] = _dequant(a_idx_ref[pl.ds(i * BM, BM), :], ca_ref[...])
    a = a_deq_ref[...]

    @pl.when(i == 0)
    def _():
        for c in range(NCHUNK):
            sl = slice(c * CH, (c + 1) * CH)
            b_deq_ref[:, sl] = _dequant(b_idx_ref[:, sl], cb_ref[...])
            obuf_ref[0, :, sl] = _band_dot(a, b_deq_ref[:, sl])

    @pl.when(i > 0)
    def _():
        obuf_ref[slot] = _band_dot(a, b_deq_ref[...])

    _out_copy(obuf_ref, hbm_out_ref, sem, i, slot).start()

    @pl.when(i == n - 1)
    def _():
        for d in range(NSLOTS - 1, -1, -1):
            @pl.when(i - d >= 0)
            def _():
                _out_copy(obuf_ref, hbm_out_ref, sem, i - d,
                          jax.lax.rem(i - d, NSLOTS)).wait()


def _run_local(a_idx, b_idx, ca, cb):
    d_out_local = a_idx.shape[0]
    return pl.pallas_call(
        _fused_kernel,
        grid=(d_out_local // BM,),
        in_specs=[
            pl.BlockSpec(memory_space=pl.ANY),
            pl.BlockSpec(memory_space=pl.ANY),
            pl.BlockSpec(memory_space=pl.ANY),
            pl.BlockSpec(memory_space=pl.ANY),
        ],
        out_specs=pl.BlockSpec(memory_space=pl.ANY),
        out_shape=jax.ShapeDtypeStruct((d_out_local, D_IN), jnp.float32),
        scratch_shapes=[
            pltpu.VMEM((d_out_local, RANK), jnp.int32),
            pltpu.VMEM((RANK, D_IN), jnp.int32),
            pltpu.VMEM((1, N_CODES), jnp.float32),
            pltpu.VMEM((1, N_CODES), jnp.float32),
            pltpu.VMEM((BM, RANK), jnp.bfloat16),
            pltpu.VMEM((RANK, D_IN), jnp.bfloat16),
            pltpu.VMEM((NSLOTS, BM, D_IN), jnp.float32),
            pltpu.SemaphoreType.DMA((NSLOTS,)),
            pltpu.SemaphoreType.DMA((4,)),
        ],
        compiler_params=pltpu.CompilerParams(
            dimension_semantics=("arbitrary",),
        ),
    )(a_idx, b_idx, ca, cb)


def kernel(A_assignments, B_assignments, A_codebook, B_codebook):
    ca = A_codebook.reshape(1, N_CODES).astype(jnp.float32)
    cb = B_codebook.reshape(1, N_CODES).astype(jnp.float32)
    return _run_local(A_assignments, B_assignments, ca, cb)
